# Initial kernel scaffold; baseline (speedup 1.0000x reference)
import functools

import jax
import jax.numpy as jnp
from jax import lax
from jax.experimental import pallas as pl
from jax.experimental.pallas import tpu as pltpu
from jax.experimental.pallas import tpu_sc as plsc

N_NODES = 10000
D_FEAT = 128
N_EDGES = 320000
R_DEC = 2.0
T_DEC = 1.0

L = 16
NC, NS = 2, 16
NW = NC * NS
EPW = N_EDGES // NW
GIDX = 80
EPC = GIDX // 2
NCH = (2 * EPW) // GIDX


def _lp_body(h_hbm, idx_hbm, out_hbm, idx_v, rows_v, sq_v, sem):
    wid = lax.axis_index("s") * NC + lax.axis_index("c")
    ebase = wid * EPW

    pltpu.sync_copy(idx_hbm.at[pl.ds(wid * NCH, NCH)], idx_v)

    def chunk_body(c, carry):
        pltpu.async_copy(h_hbm.at[idx_v.at[c]], rows_v, sem).wait()

        def edge_body(m, carry2):
            acc = jnp.zeros((L,), jnp.float32)
            for d in range(D_FEAT // L):
                a = rows_v[2 * m, pl.ds(d * L, L)]
                b = rows_v[2 * m + 1, pl.ds(d * L, L)]
                t = a - b
                acc = acc + t * t
            sq_v[c * EPC + m] = jnp.sum(acc)
            return carry2

        lax.fori_loop(0, EPC, edge_body, 0)
        return carry

    lax.fori_loop(0, NCH, chunk_body, 0)

    inv_t = 1.0 / T_DEC

    def sig_body(t, carry):
        v = sq_v[pl.ds(t * L, L)]
        sq_v[pl.ds(t * L, L)] = 1.0 / (jnp.exp((v - R_DEC) * inv_t) + 1.0)
        return carry

    lax.fori_loop(0, EPW // L, sig_body, 0)
    pltpu.sync_copy(sq_v, out_hbm.at[pl.ds(ebase, EPW)])


@jax.jit
def _lp_call(h, idx2d):
    mesh = plsc.VectorSubcoreMesh(core_axis_name="c", subcore_axis_name="s")
    fn = functools.partial(
        pl.kernel,
        out_type=jax.ShapeDtypeStruct((N_EDGES,), jnp.float32),
        mesh=mesh,
        scratch_types=[
            pltpu.VMEM((NCH, GIDX), jnp.int32),
            pltpu.VMEM((GIDX, D_FEAT), jnp.float32),
            pltpu.VMEM((EPW,), jnp.float32),
            pltpu.SemaphoreType.DMA,
        ],
    )(_lp_body)
    return fn(h, idx2d)


def kernel(h, idx):
    idx2d = idx.astype(jnp.int32).reshape(NW * NCH, GIDX)
    return _lp_call(h, idx2d)


# speedup vs baseline: 2.1285x; 2.1285x over previous
import functools

import jax
import jax.numpy as jnp
from jax import lax
from jax.experimental import pallas as pl
from jax.experimental.pallas import tpu as pltpu
from jax.experimental.pallas import tpu_sc as plsc

N_NODES = 10000
D_FEAT = 128
N_EDGES = 320000
R_DEC = 2.0
T_DEC = 1.0

L = 16
NC, NS = 2, 16
NW = NC * NS
EPW = N_EDGES // NW
GIDX = 80
EPC = GIDX // 2
NCH = (2 * EPW) // GIDX
GRP = 8


def _lp_body(h_hbm, idx_hbm, out_hbm, idx_v, rows_v, sq_v, scr_v, sem):
    wid = lax.axis_index("s") * NC + lax.axis_index("c")
    ebase = wid * EPW

    pltpu.sync_copy(idx_hbm.at[wid], idx_v)

    lanes = lax.iota(jnp.int32, L)
    lane_lo = lanes < GRP

    def chunk_body(c, carry):
        pltpu.async_copy(h_hbm.at[idx_v.at[c]], rows_v, sem).wait()

        def group_body(g, carry2):
            for l in range(GRP):
                m = g * GRP + l
                acc0 = rows_v[2 * m, pl.ds(0, L)] - rows_v[2 * m + 1, pl.ds(0, L)]
                acc = acc0 * acc0
                for d in range(1, D_FEAT // L):
                    t = (rows_v[2 * m, pl.ds(d * L, L)]
                         - rows_v[2 * m + 1, pl.ds(d * L, L)])
                    acc = acc + t * t
                scr_v[l] = acc
            s = jnp.zeros((L,), jnp.float32)
            for k in range(L):
                s = s + plsc.load_gather(
                    scr_v, [lanes, jnp.full((L,), k, jnp.int32)])
            pos = jnp.full((L,), c * EPC + g * GRP, jnp.int32) + lanes
            plsc.store_scatter(sq_v, [pos], s, mask=lane_lo)
            return carry2

        lax.fori_loop(0, EPC // GRP, group_body, 0)
        return carry

    lax.fori_loop(0, NCH, chunk_body, 0)

    inv_t = 1.0 / T_DEC

    def sig_body(t, carry):
        v = sq_v[pl.ds(t * L, L)]
        sq_v[pl.ds(t * L, L)] = 1.0 / (jnp.exp((v - R_DEC) * inv_t) + 1.0)
        return carry

    lax.fori_loop(0, EPW // L, sig_body, 0)
    pltpu.sync_copy(sq_v, out_hbm.at[pl.ds(ebase, EPW)])


@jax.jit
def _lp_call(h, idx2d):
    mesh = plsc.VectorSubcoreMesh(core_axis_name="c", subcore_axis_name="s")
    fn = functools.partial(
        pl.kernel,
        out_type=jax.ShapeDtypeStruct((N_EDGES,), jnp.float32),
        mesh=mesh,
        compiler_params=pltpu.CompilerParams(needs_layout_passes=False),
        scratch_types=[
            pltpu.VMEM((NCH, GIDX), jnp.int32),
            pltpu.VMEM((GIDX, D_FEAT), jnp.float32),
            pltpu.VMEM((EPW,), jnp.float32),
            pltpu.VMEM((L, L), jnp.float32),
            pltpu.SemaphoreType.DMA,
        ],
    )(_lp_body)
    return fn(h, idx2d)


def kernel(h, idx):
    idx2d = idx.astype(jnp.int32).reshape(NW, NCH, GIDX)
    return _lp_call(h, idx2d)


# speedup vs baseline: 3.1664x; 1.4876x over previous
import functools

import jax
import jax.numpy as jnp
from jax import lax
from jax.experimental import pallas as pl
from jax.experimental.pallas import tpu as pltpu
from jax.experimental.pallas import tpu_sc as plsc

N_NODES = 10000
D_FEAT = 128
N_EDGES = 320000
R_DEC = 2.0
T_DEC = 1.0

L = 16
NC, NS = 2, 16
NW = NC * NS
EPW = N_EDGES // NW
GIDX = 80
EPC = GIDX // 2
NCH = (2 * EPW) // GIDX
GRP = 8


def _lp_body(h_hbm, idx_hbm, out_hbm, idx_v, rows0_v, rows1_v, sq_v, scr_v,
             sem0, sem1):
    wid = lax.axis_index("s") * NC + lax.axis_index("c")
    ebase = wid * EPW

    pltpu.sync_copy(idx_hbm.at[wid], idx_v)

    lanes = lax.iota(jnp.int32, L)
    lane_lo = lanes < GRP

    def start(c, rows, sem):
        pltpu.async_copy(h_hbm.at[idx_v.at[c]], rows, sem)

    def drain(rows, sem):
        pltpu.make_async_copy(h_hbm.at[idx_v.at[0]], rows, sem).wait()

    def compute(c, rows):
        def group_body(g, carry):
            for l in range(GRP):
                m = g * GRP + l
                acc0 = rows[2 * m, pl.ds(0, L)] - rows[2 * m + 1, pl.ds(0, L)]
                acc = acc0 * acc0
                for d in range(1, D_FEAT // L):
                    t = (rows[2 * m, pl.ds(d * L, L)]
                         - rows[2 * m + 1, pl.ds(d * L, L)])
                    acc = acc + t * t
                scr_v[l] = acc
            s = jnp.zeros((L,), jnp.float32)
            for k in range(L):
                s = s + plsc.load_gather(
                    scr_v, [lanes, jnp.full((L,), k, jnp.int32)])
            pos = jnp.full((L,), c * EPC + g * GRP, jnp.int32) + lanes
            plsc.store_scatter(sq_v, [pos], s, mask=lane_lo)
            return carry

        lax.fori_loop(0, EPC // GRP, group_body, 0)

    start(0, rows0_v, sem0)

    def pair_body(p, carry):
        c0 = 2 * p
        start(c0 + 1, rows1_v, sem1)
        drain(rows0_v, sem0)
        compute(c0, rows0_v)

        @pl.when(c0 + 2 < NCH)
        def _():
            start(c0 + 2, rows0_v, sem0)

        drain(rows1_v, sem1)
        compute(c0 + 1, rows1_v)
        return carry

    lax.fori_loop(0, NCH // 2, pair_body, 0)

    inv_t = 1.0 / T_DEC

    def sig_body(t, carry):
        v = sq_v[pl.ds(t * L, L)]
        sq_v[pl.ds(t * L, L)] = 1.0 / (jnp.exp((v - R_DEC) * inv_t) + 1.0)
        return carry

    lax.fori_loop(0, EPW // L, sig_body, 0)
    pltpu.sync_copy(sq_v, out_hbm.at[pl.ds(ebase, EPW)])


@jax.jit
def _lp_call(h, idx2d):
    mesh = plsc.VectorSubcoreMesh(core_axis_name="c", subcore_axis_name="s")
    fn = functools.partial(
        pl.kernel,
        out_type=jax.ShapeDtypeStruct((N_EDGES,), jnp.float32),
        mesh=mesh,
        compiler_params=pltpu.CompilerParams(needs_layout_passes=False),
        scratch_types=[
            pltpu.VMEM((NCH, GIDX), jnp.int32),
            pltpu.VMEM((GIDX, D_FEAT), jnp.float32),
            pltpu.VMEM((GIDX, D_FEAT), jnp.float32),
            pltpu.VMEM((EPW,), jnp.float32),
            pltpu.VMEM((L, L), jnp.float32),
            pltpu.SemaphoreType.DMA,
            pltpu.SemaphoreType.DMA,
        ],
    )(_lp_body)
    return fn(h, idx2d)


def kernel(h, idx):
    idx2d = idx.astype(jnp.int32).reshape(NW, NCH, GIDX)
    return _lp_call(h, idx2d)


# speedup vs baseline: 3.7319x; 1.1786x over previous
import functools

import jax
import jax.numpy as jnp
from jax import lax
from jax.experimental import pallas as pl
from jax.experimental.pallas import tpu as pltpu
from jax.experimental.pallas import tpu_sc as plsc

N_NODES = 10000
D_FEAT = 128
N_EDGES = 320000
R_DEC = 2.0
T_DEC = 1.0

L = 16
NC, NS = 2, 16
NW = NC * NS
EPW = N_EDGES // NW
GIDX = 80
EPC = GIDX // 2
NCH = (2 * EPW) // GIDX
GRP = 8


def _lp_body(h_hbm, idx_hbm, out_hbm, idx_v, rows0_v, rows1_v, sq_v, scr_v,
             sem0, sem1):
    wid = lax.axis_index("s") * NC + lax.axis_index("c")
    ebase = wid * EPW

    pltpu.sync_copy(idx_hbm.at[wid], idx_v)

    lanes = lax.iota(jnp.int32, L)
    lane_lo = lanes < GRP

    def start(c, rows, sem):
        pltpu.async_copy(h_hbm.at[idx_v.at[c]], rows, sem)

    def drain(rows, sem):
        pltpu.make_async_copy(h_hbm.at[idx_v.at[0]], rows, sem).wait()

    def compute(c, rows):
        def group_body(g, carry):
            for l in range(GRP):
                m = g * GRP + l
                acc = jnp.zeros((L,), jnp.float32)
                for d in range(D_FEAT // (2 * L)):
                    a = plsc.bitcast(rows[2 * m, pl.ds(d * L, L)],
                                     jnp.bfloat16)
                    b = plsc.bitcast(rows[2 * m + 1, pl.ds(d * L, L)],
                                     jnp.bfloat16)
                    t0, t1 = plsc.unpack(a - b,
                                         format=plsc.PackFormat.INTERLEAVED)
                    acc = acc + t0 * t0 + t1 * t1
                scr_v[l] = acc
            s = jnp.zeros((L,), jnp.float32)
            for k in range(L):
                s = s + plsc.load_gather(
                    scr_v, [lanes, jnp.full((L,), k, jnp.int32)])
            pos = jnp.full((L,), c * EPC + g * GRP, jnp.int32) + lanes
            plsc.store_scatter(sq_v, [pos], s, mask=lane_lo)
            return carry

        lax.fori_loop(0, EPC // GRP, group_body, 0)

    start(0, rows0_v, sem0)

    def pair_body(p, carry):
        c0 = 2 * p
        start(c0 + 1, rows1_v, sem1)
        drain(rows0_v, sem0)
        compute(c0, rows0_v)

        @pl.when(c0 + 2 < NCH)
        def _():
            start(c0 + 2, rows0_v, sem0)

        drain(rows1_v, sem1)
        compute(c0 + 1, rows1_v)
        return carry

    lax.fori_loop(0, NCH // 2, pair_body, 0)

    inv_t = 1.0 / T_DEC

    def sig_body(t, carry):
        v = sq_v[pl.ds(t * L, L)]
        sq_v[pl.ds(t * L, L)] = 1.0 / (jnp.exp((v - R_DEC) * inv_t) + 1.0)
        return carry

    lax.fori_loop(0, EPW // L, sig_body, 0)
    pltpu.sync_copy(sq_v, out_hbm.at[pl.ds(ebase, EPW)])


@jax.jit
def _lp_call(h, idx2d):
    mesh = plsc.VectorSubcoreMesh(core_axis_name="c", subcore_axis_name="s")
    fn = functools.partial(
        pl.kernel,
        out_type=jax.ShapeDtypeStruct((N_EDGES,), jnp.float32),
        mesh=mesh,
        compiler_params=pltpu.CompilerParams(needs_layout_passes=False,
                                             use_tc_tiling_on_sc=False),
        scratch_types=[
            pltpu.VMEM((NCH, GIDX), jnp.int32),
            pltpu.VMEM((GIDX, D_FEAT // 2), jnp.int32),
            pltpu.VMEM((GIDX, D_FEAT // 2), jnp.int32),
            pltpu.VMEM((EPW,), jnp.float32),
            pltpu.VMEM((L, L), jnp.float32),
            pltpu.SemaphoreType.DMA,
            pltpu.SemaphoreType.DMA,
        ],
    )(_lp_body)
    return fn(h, idx2d)


def kernel(h, idx):
    hp = lax.bitcast_convert_type(
        h.astype(jnp.bfloat16).reshape(N_NODES, D_FEAT // 2, 2), jnp.int32)
    idx2d = idx.astype(jnp.int32).reshape(NW, NCH, GIDX)
    return _lp_call(hp, idx2d)


# speedup vs baseline: 3.7539x; 1.0059x over previous
import functools

import jax
import jax.numpy as jnp
from jax import lax
from jax.experimental import pallas as pl
from jax.experimental.pallas import tpu as pltpu
from jax.experimental.pallas import tpu_sc as plsc

N_NODES = 10000
D_FEAT = 128
N_EDGES = 320000
R_DEC = 2.0
T_DEC = 1.0

L = 16
NC, NS = 2, 16
NW = NC * NS
EPW = N_EDGES // NW
GIDX = 80
EPC = GIDX // 2
NCH = (2 * EPW) // GIDX
GRP = 8
NBUF = 4


def _lp_body(h_hbm, idx_hbm, out_hbm, idx_v, rows0_v, rows1_v, rows2_v,
             rows3_v, sq_v, scr_v, sem0, sem1, sem2, sem3):
    wid = lax.axis_index("s") * NC + lax.axis_index("c")
    ebase = wid * EPW

    pltpu.sync_copy(idx_hbm.at[wid], idx_v)

    lanes = lax.iota(jnp.int32, L)
    lane_lo = lanes < GRP

    def start(c, rows, sem):
        pltpu.async_copy(h_hbm.at[idx_v.at[c]], rows, sem)

    def drain(rows, sem):
        pltpu.make_async_copy(h_hbm.at[idx_v.at[0]], rows, sem).wait()

    def compute(c, rows):
        def group_body(g, carry):
            for l in range(GRP):
                m = g * GRP + l
                acc = jnp.zeros((L,), jnp.float32)
                for d in range(D_FEAT // (2 * L)):
                    a = plsc.bitcast(rows[2 * m, pl.ds(d * L, L)],
                                     jnp.bfloat16)
                    b = plsc.bitcast(rows[2 * m + 1, pl.ds(d * L, L)],
                                     jnp.bfloat16)
                    t0, t1 = plsc.unpack(a - b,
                                         format=plsc.PackFormat.INTERLEAVED)
                    acc = acc + t0 * t0 + t1 * t1
                scr_v[l] = acc
            s = jnp.zeros((L,), jnp.float32)
            for k in range(L):
                s = s + plsc.load_gather(
                    scr_v, [lanes, jnp.full((L,), k, jnp.int32)])
            pos = jnp.full((L,), c * EPC + g * GRP, jnp.int32) + lanes
            plsc.store_scatter(sq_v, [pos], s, mask=lane_lo)
            return carry

        lax.fori_loop(0, EPC // GRP, group_body, 0)

    bufs = [(rows0_v, sem0), (rows1_v, sem1), (rows2_v, sem2), (rows3_v, sem3)]
    for i in range(NBUF):
        start(i, *bufs[i])

    def quad_body(p, carry):
        c0 = NBUF * p
        for i in range(NBUF):
            rows, sem = bufs[i]
            drain(rows, sem)
            compute(c0 + i, rows)

            @pl.when(c0 + NBUF + i < NCH)
            def _():
                start(c0 + NBUF + i, rows, sem)

        return carry

    lax.fori_loop(0, NCH // NBUF, quad_body, 0)
    for i in range(NCH % NBUF):
        rows, sem = bufs[i]
        drain(rows, sem)
        compute(NBUF * (NCH // NBUF) + i, rows)

    inv_t = 1.0 / T_DEC

    def sig_body(t, carry):
        v = sq_v[pl.ds(t * L, L)]
        sq_v[pl.ds(t * L, L)] = 1.0 / (jnp.exp((v - R_DEC) * inv_t) + 1.0)
        return carry

    lax.fori_loop(0, EPW // L, sig_body, 0)
    pltpu.sync_copy(sq_v, out_hbm.at[pl.ds(ebase, EPW)])


@jax.jit
def _lp_call(h, idx2d):
    mesh = plsc.VectorSubcoreMesh(core_axis_name="c", subcore_axis_name="s")
    fn = functools.partial(
        pl.kernel,
        out_type=jax.ShapeDtypeStruct((N_EDGES,), jnp.float32),
        mesh=mesh,
        compiler_params=pltpu.CompilerParams(needs_layout_passes=False,
                                             use_tc_tiling_on_sc=False),
        scratch_types=[
            pltpu.VMEM((NCH, GIDX), jnp.int32),
            pltpu.VMEM((GIDX, D_FEAT // 2), jnp.int32),
            pltpu.VMEM((GIDX, D_FEAT // 2), jnp.int32),
            pltpu.VMEM((GIDX, D_FEAT // 2), jnp.int32),
            pltpu.VMEM((GIDX, D_FEAT // 2), jnp.int32),
            pltpu.VMEM((EPW,), jnp.float32),
            pltpu.VMEM((L, L), jnp.float32),
            pltpu.SemaphoreType.DMA,
            pltpu.SemaphoreType.DMA,
            pltpu.SemaphoreType.DMA,
            pltpu.SemaphoreType.DMA,
        ],
    )(_lp_body)
    return fn(h, idx2d)


def kernel(h, idx):
    hp = lax.bitcast_convert_type(
        h.astype(jnp.bfloat16).reshape(N_NODES, D_FEAT // 2, 2), jnp.int32)
    idx2d = idx.astype(jnp.int32).reshape(NW, NCH, GIDX)
    return _lp_call(hp, idx2d)
